# baseline (speedup 1.0000x reference)
import jax
import jax.numpy as jnp
from jax import lax
from jax.experimental import pallas as pl
from jax.experimental.pallas import tpu as pltpu
from jax.experimental.pallas import tpu_sc as plsc

NUM_EMBEDDINGS = 1024
EMBEDDING_DIM = 256

_NC = 2
_NS = 16
_NW = _NC * _NS
_L = 16

_PIX_BLOCK = 8192


def _argmin_body(z_ref, w_ref, idx_ref):
    zb = z_ref[...]
    w = w_ref[...]
    m = lax.dot_general(zb, w, (((1,), (1,)), ((), ())),
                        preferred_element_type=jnp.float32)
    zsq = jnp.sum(zb * zb, axis=1, keepdims=True)
    wsq = jnp.sum(w * w, axis=1)[None, :]
    dist = (zsq - 2.0 * m) + wsq
    mn = jnp.min(dist, axis=1, keepdims=True)
    k_iota = lax.broadcasted_iota(jnp.int32, dist.shape, 1)
    cand = jnp.where(dist == mn, k_iota, NUM_EMBEDDINGS)
    idx_ref[0, 0, :] = jnp.min(cand, axis=1)


def _tc_argmin(z_flat, w):
    n_pix = z_flat.shape[0]
    grid = n_pix // _PIX_BLOCK
    out = pl.pallas_call(
        _argmin_body,
        grid=(grid,),
        in_specs=[
            pl.BlockSpec((_PIX_BLOCK, EMBEDDING_DIM), lambda i: (i, 0)),
            pl.BlockSpec((NUM_EMBEDDINGS, EMBEDDING_DIM), lambda i: (0, 0)),
        ],
        out_specs=pl.BlockSpec((1, 1, _PIX_BLOCK), lambda i: (i, 0, 0)),
        out_shape=jax.ShapeDtypeStruct((grid, 1, _PIX_BLOCK), jnp.int32),
    )(z_flat, w)
    return out.reshape(-1)


_CHUNK = 64
_NCHUNK = 4


def _sc_body(w_hbm, idx_hbm, z_hbm, q_hbm, loss_hbm,
             idx_v, rows_v, z_v, part_v, gsem, zsem):
    wid = lax.axis_index("s") * _NC + lax.axis_index("c")
    rows_per_w = 8192 // _NW
    base = wid * rows_per_w

    def start(c, buf):
        row0 = base + c * _CHUNK
        pltpu.sync_copy(idx_hbm.at[pl.ds(row0, _CHUNK)], idx_v.at[buf])
        pltpu.async_copy(w_hbm.at[idx_v.at[buf]], rows_v.at[buf], gsem)
        pltpu.async_copy(z_hbm.at[pl.ds(row0, _CHUNK)], z_v.at[buf], zsem)

    acc0 = jnp.zeros((_L,), jnp.float32)
    start(0, 0)
    for c in range(_NCHUNK):
        buf = c % 2
        pltpu.make_async_copy(w_hbm.at[idx_v.at[buf]], rows_v.at[buf],
                              gsem).wait()
        pltpu.make_async_copy(z_hbm.at[pl.ds(0, _CHUNK)], z_v.at[buf],
                              zsem).wait()
        if c + 1 < _NCHUNK:
            start(c + 1, 1 - buf)

        def body(r, acc):
            for l in range(EMBEDDING_DIM // _L):
                sl = pl.ds(l * _L, _L)
                d = rows_v[buf, r, sl] - z_v[buf, r, sl]
                rows_v[buf, r, sl] = z_v[buf, r, sl] + d
                acc = acc + d * d
            return acc

        acc0 = lax.fori_loop(0, _CHUNK, body, acc0)
        pltpu.sync_copy(rows_v.at[buf], q_hbm.at[pl.ds(base + c * _CHUNK, _CHUNK)])

    part_v[0, :] = acc0
    pltpu.sync_copy(part_v, loss_hbm.at[pl.ds(wid, 1)])


def _sc_gather_loss(w, idx_flat, z_nat):
    mesh = plsc.VectorSubcoreMesh(core_axis_name="c", subcore_axis_name="s")
    kfn = pl.kernel(
        _sc_body,
        out_type=(
            jax.ShapeDtypeStruct((8192, EMBEDDING_DIM), jnp.float32),
            jax.ShapeDtypeStruct((_NW, _L), jnp.float32),
        ),
        mesh=mesh,
        scratch_types=[
            pltpu.VMEM((2, _CHUNK), jnp.int32),
            pltpu.VMEM((2, _CHUNK, EMBEDDING_DIM), jnp.float32),
            pltpu.VMEM((2, _CHUNK, EMBEDDING_DIM), jnp.float32),
            pltpu.VMEM((1, _L), jnp.float32),
            pltpu.SemaphoreType.DMA,
            pltpu.SemaphoreType.DMA,
        ],
    )
    return kfn(w, idx_flat, z_nat)


def kernel(z, W):
    B, C, H, Wd = z.shape
    n = B * C * H * Wd
    z_flat = jnp.transpose(z, (0, 2, 3, 1)).reshape(-1, C)
    idx_flat = _tc_argmin(z_flat, W)
    z_nat = z.reshape(-1, C)
    q_flat, partials = _sc_gather_loss(W, idx_flat, z_nat)
    quantized_st = q_flat.reshape(z.shape)
    m = jnp.sum(partials) / n
    loss = 0.25 * m + m
    indices = idx_flat.reshape(B, H, Wd)
    return (quantized_st, indices, loss)


# speedup vs baseline: 1.0088x; 1.0088x over previous
import jax
import jax.numpy as jnp
from jax import lax
from jax.experimental import pallas as pl
from jax.experimental.pallas import tpu as pltpu
from jax.experimental.pallas import tpu_sc as plsc

NUM_EMBEDDINGS = 1024
EMBEDDING_DIM = 256

_NC = 2
_NS = 16
_NW = _NC * _NS
_L = 16

_PIX_BLOCK = 4096


def _argmin_body(z_ref, w_ref, idx_ref):
    zb = z_ref[...]
    w = w_ref[...]
    m = lax.dot_general(zb, w, (((1,), (1,)), ((), ())),
                        preferred_element_type=jnp.float32)
    zsq = jnp.sum(zb * zb, axis=1, keepdims=True)
    wsq = jnp.sum(w * w, axis=1)[None, :]
    dist = (zsq - 2.0 * m) + wsq
    mn = jnp.min(dist, axis=1, keepdims=True)
    k_iota = lax.broadcasted_iota(jnp.int32, dist.shape, 1)
    cand = jnp.where(dist == mn, k_iota, NUM_EMBEDDINGS)
    idx_ref[0, 0, :] = jnp.min(cand, axis=1)


def _tc_argmin(z_flat, w):
    n_pix = z_flat.shape[0]
    grid = n_pix // _PIX_BLOCK
    out = pl.pallas_call(
        _argmin_body,
        grid=(grid,),
        in_specs=[
            pl.BlockSpec((_PIX_BLOCK, EMBEDDING_DIM), lambda i: (i, 0)),
            pl.BlockSpec((NUM_EMBEDDINGS, EMBEDDING_DIM), lambda i: (0, 0)),
        ],
        out_specs=pl.BlockSpec((1, 1, _PIX_BLOCK), lambda i: (i, 0, 0)),
        out_shape=jax.ShapeDtypeStruct((grid, 1, _PIX_BLOCK), jnp.int32),
    )(z_flat, w)
    return out.reshape(-1)


_CHUNK = 64
_NCHUNK = 4


def _sc_body(w_hbm, idx_hbm, z_hbm, q_hbm, loss_hbm,
             idx_v, rows_v, z_v, part_v, gsem, zsem):
    wid = lax.axis_index("s") * _NC + lax.axis_index("c")
    rows_per_w = 8192 // _NW
    base = wid * rows_per_w

    def start(c, buf):
        row0 = base + c * _CHUNK
        pltpu.sync_copy(idx_hbm.at[pl.ds(row0, _CHUNK)], idx_v.at[buf])
        pltpu.async_copy(w_hbm.at[idx_v.at[buf]], rows_v.at[buf], gsem)
        pltpu.async_copy(z_hbm.at[pl.ds(row0, _CHUNK)], z_v.at[buf], zsem)

    acc0 = jnp.zeros((_L,), jnp.float32)
    start(0, 0)
    for c in range(_NCHUNK):
        buf = c % 2
        pltpu.make_async_copy(w_hbm.at[idx_v.at[buf]], rows_v.at[buf],
                              gsem).wait()
        pltpu.make_async_copy(z_hbm.at[pl.ds(0, _CHUNK)], z_v.at[buf],
                              zsem).wait()
        if c + 1 < _NCHUNK:
            start(c + 1, 1 - buf)

        def body(r, acc):
            for l in range(EMBEDDING_DIM // _L):
                sl = pl.ds(l * _L, _L)
                d = rows_v[buf, r, sl] - z_v[buf, r, sl]
                rows_v[buf, r, sl] = z_v[buf, r, sl] + d
                acc = acc + d * d
            return acc

        acc0 = lax.fori_loop(0, _CHUNK, body, acc0)
        pltpu.sync_copy(rows_v.at[buf], q_hbm.at[pl.ds(base + c * _CHUNK, _CHUNK)])

    part_v[0, :] = acc0
    pltpu.sync_copy(part_v, loss_hbm.at[pl.ds(wid, 1)])


def _sc_gather_loss(w, idx_flat, z_nat):
    mesh = plsc.VectorSubcoreMesh(core_axis_name="c", subcore_axis_name="s")
    kfn = pl.kernel(
        _sc_body,
        out_type=(
            jax.ShapeDtypeStruct((8192, EMBEDDING_DIM), jnp.float32),
            jax.ShapeDtypeStruct((_NW, _L), jnp.float32),
        ),
        mesh=mesh,
        scratch_types=[
            pltpu.VMEM((2, _CHUNK), jnp.int32),
            pltpu.VMEM((2, _CHUNK, EMBEDDING_DIM), jnp.float32),
            pltpu.VMEM((2, _CHUNK, EMBEDDING_DIM), jnp.float32),
            pltpu.VMEM((1, _L), jnp.float32),
            pltpu.SemaphoreType.DMA,
            pltpu.SemaphoreType.DMA,
        ],
    )
    return kfn(w, idx_flat, z_nat)


def kernel(z, W):
    B, C, H, Wd = z.shape
    n = B * C * H * Wd
    z_flat = jnp.transpose(z, (0, 2, 3, 1)).reshape(-1, C)
    idx_flat = _tc_argmin(z_flat, W)
    z_nat = z.reshape(-1, C)
    q_flat, partials = _sc_gather_loss(W, idx_flat, z_nat)
    quantized_st = q_flat.reshape(z.shape)
    m = jnp.sum(partials) / n
    loss = 0.25 * m + m
    indices = idx_flat.reshape(B, H, Wd)
    return (quantized_st, indices, loss)
